# Initial kernel scaffold; baseline (speedup 1.0000x reference)
import jax
import jax.numpy as jnp
from jax.experimental import pallas as pl


def kernel(xyz, points, W1, b1, g1, be1, W2, b2, g2, be2, W3, b3, g3, be3):
    raise NotImplementedError("write your pallas kernel here")


# speedup vs baseline: 215.0253x; 215.0253x over previous
import functools

import jax
import jax.numpy as jnp
from jax import lax
from jax.experimental import pallas as pl
from jax.experimental.pallas import tpu as pltpu
from jax.experimental.pallas import tpu_sc as plsc

B = 8
N = 4096
S = 512
NS = 32
R2 = 0.2 ** 2
DPAD = 128
NTILES = 32
S_PER_TILE = S * B // NTILES
NCHUNK = N // 16


def _r(v):
    return v.astype(jnp.bfloat16).astype(jnp.float32)


def _fps_body(xyz_ref, nxyz_ref, nxyzb_ref, cc_ref, xx_ref, xyzb_ref):
    x = xyz_ref[:, 0, :]
    y = xyz_ref[:, 1, :]
    z = xyz_ref[:, 2, :]
    xx_ref[...] = (x * x + y * y) + z * z
    xyzb_ref[0] = _r(x)
    xyzb_ref[1] = _r(y)
    xyzb_ref[2] = _r(z)
    iota = lax.broadcasted_iota(jnp.int32, (B, N), 1)
    iota_s = lax.broadcasted_iota(jnp.int32, (B, S), 1)

    def body(i, st):
        dist, far, nx, ny, nz = st
        sel = iota == far
        cx = jnp.sum(jnp.where(sel, x, 0.0), axis=1, keepdims=True)
        cy = jnp.sum(jnp.where(sel, y, 0.0), axis=1, keepdims=True)
        cz = jnp.sum(jnp.where(sel, z, 0.0), axis=1, keepdims=True)
        sel_s = iota_s == i
        nx = jnp.where(sel_s, cx, nx)
        ny = jnp.where(sel_s, cy, ny)
        nz = jnp.where(sel_s, cz, nz)
        dx, dy, dz = x - cx, y - cy, z - cz
        d = (dx * dx + dy * dy) + dz * dz
        dist = jnp.minimum(dist, d)
        mx = jnp.max(dist, axis=1, keepdims=True)
        far = jnp.min(jnp.where(dist == mx, iota, N), axis=1, keepdims=True)
        return dist, far, nx, ny, nz

    dist0 = jnp.full((B, N), 1e10, jnp.float32)
    far0 = jnp.zeros((B, 1), jnp.int32)
    z0 = jnp.zeros((B, S), jnp.float32)
    _, _, nx, ny, nz = lax.fori_loop(0, S, body, (dist0, far0, z0, z0, z0))
    nxyz_ref[0] = nx
    nxyz_ref[1] = ny
    nxyz_ref[2] = nz
    nxyzb_ref[0] = _r(nx)
    nxyzb_ref[1] = _r(ny)
    nxyzb_ref[2] = _r(nz)
    cc_ref[...] = (nx * nx + ny * ny) + nz * nz


def _fps(xyz):
    return pl.pallas_call(
        _fps_body,
        out_shape=(
            jax.ShapeDtypeStruct((3, B, S), jnp.float32),
            jax.ShapeDtypeStruct((3, B, S), jnp.float32),
            jax.ShapeDtypeStruct((B, S), jnp.float32),
            jax.ShapeDtypeStruct((B, N), jnp.float32),
            jax.ShapeDtypeStruct((3, B, N), jnp.float32),
        ),
    )(xyz)


def _sc_group_body(xyzb_hbm, xx_hbm, nxyzb_hbm, cc_hbm, f_hbm, out_hbm,
                   xv, yv, zv, xxv, cxv, cyv, czv, ccv, idxbuf, idxsel, gbuf,
                   sem):
    cid = lax.axis_index("c")
    sid = lax.axis_index("s")
    wid = sid * 2 + cid
    b = wid // (NTILES // B)
    q = wid % (NTILES // B)
    s0 = q * S_PER_TILE

    pltpu.sync_copy(xyzb_hbm.at[pl.ds((0 * B + b) * N, N)], xv)
    pltpu.sync_copy(xyzb_hbm.at[pl.ds((1 * B + b) * N, N)], yv)
    pltpu.sync_copy(xyzb_hbm.at[pl.ds((2 * B + b) * N, N)], zv)
    pltpu.sync_copy(xx_hbm.at[pl.ds(b * N, N)], xxv)
    pltpu.sync_copy(nxyzb_hbm.at[pl.ds((0 * B + b) * S + s0, S_PER_TILE)], cxv)
    pltpu.sync_copy(nxyzb_hbm.at[pl.ds((1 * B + b) * S + s0, S_PER_TILE)], cyv)
    pltpu.sync_copy(nxyzb_hbm.at[pl.ds((2 * B + b) * S + s0, S_PER_TILE)], czv)
    pltpu.sync_copy(cc_hbm.at[pl.ds(b * S + s0, S_PER_TILE)], ccv)

    lane = lax.iota(jnp.int32, 16)

    def per_centroid(i, carry):
        ii = jnp.full((16,), i, jnp.int32)
        cxs = plsc.load_gather(cxv, [ii])
        cys = plsc.load_gather(cyv, [ii])
        czs = plsc.load_gather(czv, [ii])
        ccs = plsc.load_gather(ccv, [ii])

        def cond(st):
            j, cnt = st
            return (j < NCHUNK) & (cnt < NS)

        def chunk(st):
            j, cnt = st
            off = j * 16
            xc = xv[pl.ds(off, 16)]
            yc = yv[pl.ds(off, 16)]
            zc = zv[pl.ds(off, 16)]
            xxc = xxv[pl.ds(off, 16)]
            dot = (cxs * xc + cys * yc) + czs * zc
            d = (ccs + xxc) - 2.0 * dot
            m = d <= R2
            mi = m.astype(jnp.int32)
            pos = plsc.cumsum(mi) + cnt
            w = m & (pos <= NS)
            plsc.store_scatter(idxbuf, [pos - 1], lane + off, mask=w)
            return j + 1, cnt + jnp.sum(mi)

        _, cnt = lax.while_loop(cond, chunk, (jnp.int32(0), jnp.int32(0)))

        first = plsc.load_gather(idxbuf, [jnp.zeros((16,), jnp.int32)])
        for h in range(NS // 16):
            v = idxbuf[pl.ds(h * 16, 16)]
            v = jnp.where(lane + h * 16 < cnt, v, first)
            idxsel[pl.ds(h * 16, 16)] = v

        pltpu.async_copy(f_hbm.at[b].at[idxsel], gbuf, sem).wait()
        pltpu.sync_copy(gbuf, out_hbm.at[b, s0 + i])
        return carry

    lax.fori_loop(0, S_PER_TILE, per_centroid, jnp.int32(0))


@functools.partial(
    pl.kernel,
    out_type=jax.ShapeDtypeStruct((B, S, NS, DPAD), jnp.float32),
    mesh=plsc.VectorSubcoreMesh(
        core_axis_name="c", subcore_axis_name="s", num_cores=2, num_subcores=16
    ),
    compiler_params=pltpu.CompilerParams(needs_layout_passes=False),
    scratch_types=[
        pltpu.VMEM((N,), jnp.float32),
        pltpu.VMEM((N,), jnp.float32),
        pltpu.VMEM((N,), jnp.float32),
        pltpu.VMEM((N,), jnp.float32),
        pltpu.VMEM((S_PER_TILE,), jnp.float32),
        pltpu.VMEM((S_PER_TILE,), jnp.float32),
        pltpu.VMEM((S_PER_TILE,), jnp.float32),
        pltpu.VMEM((S_PER_TILE,), jnp.float32),
        pltpu.VMEM((NS,), jnp.int32),
        pltpu.VMEM((NS,), jnp.int32),
        pltpu.VMEM((NS, DPAD), jnp.float32),
        pltpu.SemaphoreType.DMA,
    ],
)
def _sc_group(*refs):
    _sc_group_body(*refs)


SBLK = 128


def _mlp_body(g_ref, c_ref, w1_ref, a1_ref, w2_ref, a2_ref, w3_ref, a3_ref,
              out_ref):
    X = g_ref[0].reshape(SBLK * NS, DPAD)
    h = jnp.dot(X, w1_ref[...], preferred_element_type=jnp.float32)
    h = h + a1_ref[...]
    corr = jnp.dot(c_ref[0], w1_ref[0:3, :], preferred_element_type=jnp.float32)
    h = h.reshape(SBLK, NS, 64) - corr[:, None, :]
    h = jnp.maximum(h, 0.0).reshape(SBLK * NS, 64)
    h = jnp.dot(h, w2_ref[...], preferred_element_type=jnp.float32) + a2_ref[...]
    h = jnp.maximum(h, 0.0)
    h = jnp.dot(h, w3_ref[...], preferred_element_type=jnp.float32) + a3_ref[...]
    h = jnp.maximum(h, 0.0)
    out_ref[0] = jnp.max(h.reshape(SBLK, NS, 128), axis=1)


def _mlp(G, c, w1, a1, w2, a2, w3, a3):
    grid = (B, S // SBLK)
    return pl.pallas_call(
        _mlp_body,
        grid=grid,
        in_specs=[
            pl.BlockSpec((1, SBLK, NS, DPAD), lambda b, s: (b, s, 0, 0)),
            pl.BlockSpec((1, SBLK, 3), lambda b, s: (b, s, 0)),
            pl.BlockSpec((DPAD, 64), lambda b, s: (0, 0)),
            pl.BlockSpec((1, 64), lambda b, s: (0, 0)),
            pl.BlockSpec((64, 64), lambda b, s: (0, 0)),
            pl.BlockSpec((1, 64), lambda b, s: (0, 0)),
            pl.BlockSpec((64, 128), lambda b, s: (0, 0)),
            pl.BlockSpec((1, 128), lambda b, s: (0, 0)),
        ],
        out_specs=pl.BlockSpec((1, SBLK, 128), lambda b, s: (b, s, 0)),
        out_shape=jax.ShapeDtypeStruct((B, S, 128), jnp.float32),
    )(G, c, w1, a1, w2, a2, w3, a3)


def kernel(xyz, points, W1, b1, g1, be1, W2, b2, g2, be2, W3, b3, g3, be3):
    nxyz, nxyzb, cc, xx, xyzb = _fps(xyz)

    xyz_t = jnp.transpose(xyz, (0, 2, 1))
    pts_t = jnp.transpose(points, (0, 2, 1))
    F = jnp.concatenate(
        [xyz_t, pts_t, jnp.zeros((B, N, DPAD - 67), jnp.float32)], axis=-1)

    G = _sc_group(xyzb.reshape(-1), xx.reshape(-1), nxyzb.reshape(-1),
                  cc.reshape(-1), F)

    w1 = jnp.pad(W1 * g1[:, None], ((0, 0), (0, DPAD - 67))).T
    a1 = (b1 * g1 + be1)[None, :]
    w2 = (W2 * g2[:, None]).T
    a2 = (b2 * g2 + be2)[None, :]
    w3 = (W3 * g3[:, None]).T
    a3 = (b3 * g3 + be3)[None, :]

    c = jnp.transpose(nxyz, (1, 2, 0))
    out = _mlp(G, c, w1, a1, w2, a2, w3, a3)

    new_xyz = jnp.transpose(nxyz, (1, 0, 2))
    new_points = jnp.transpose(out, (0, 2, 1))
    return new_xyz, new_points


# speedup vs baseline: 243.7665x; 1.1337x over previous
import functools

import jax
import jax.numpy as jnp
from jax import lax
from jax.experimental import pallas as pl
from jax.experimental.pallas import tpu as pltpu
from jax.experimental.pallas import tpu_sc as plsc

B = 8
N = 4096
S = 512
NS = 32
R2 = 0.2 ** 2
DPAD = 128
NTILES = 32
S_PER_TILE = S * B // NTILES
NCHUNK = N // 16


def _r(v):
    return v.astype(jnp.bfloat16).astype(jnp.float32)


def _fps_body(xyz_ref, nxyz_ref, nxyzb_ref, cc_ref, xx_ref, xyzb_ref):
    x = xyz_ref[:, 0, :]
    y = xyz_ref[:, 1, :]
    z = xyz_ref[:, 2, :]
    xx_ref[...] = (x * x + y * y) + z * z
    xyzb_ref[0] = _r(x)
    xyzb_ref[1] = _r(y)
    xyzb_ref[2] = _r(z)
    iota = lax.broadcasted_iota(jnp.int32, (B, N), 1)
    iota_s = lax.broadcasted_iota(jnp.int32, (B, S), 1)

    def body(i, st):
        dist, far, nx, ny, nz = st
        sel = iota == far
        cx = jnp.sum(jnp.where(sel, x, 0.0), axis=1, keepdims=True)
        cy = jnp.sum(jnp.where(sel, y, 0.0), axis=1, keepdims=True)
        cz = jnp.sum(jnp.where(sel, z, 0.0), axis=1, keepdims=True)
        sel_s = iota_s == i
        nx = jnp.where(sel_s, cx, nx)
        ny = jnp.where(sel_s, cy, ny)
        nz = jnp.where(sel_s, cz, nz)
        dx, dy, dz = x - cx, y - cy, z - cz
        d = (dx * dx + dy * dy) + dz * dz
        dist = jnp.minimum(dist, d)
        mx = jnp.max(dist, axis=1, keepdims=True)
        far = jnp.min(jnp.where(dist == mx, iota, N), axis=1, keepdims=True)
        return dist, far, nx, ny, nz

    dist0 = jnp.full((B, N), 1e10, jnp.float32)
    far0 = jnp.zeros((B, 1), jnp.int32)
    z0 = jnp.zeros((B, S), jnp.float32)
    _, _, nx, ny, nz = lax.fori_loop(0, S, body, (dist0, far0, z0, z0, z0))
    nxyz_ref[0] = nx
    nxyz_ref[1] = ny
    nxyz_ref[2] = nz
    nxyzb_ref[0] = _r(nx)
    nxyzb_ref[1] = _r(ny)
    nxyzb_ref[2] = _r(nz)
    cc_ref[...] = (nx * nx + ny * ny) + nz * nz


def _fps(xyz):
    return pl.pallas_call(
        _fps_body,
        out_shape=(
            jax.ShapeDtypeStruct((3, B, S), jnp.float32),
            jax.ShapeDtypeStruct((3, B, S), jnp.float32),
            jax.ShapeDtypeStruct((B, S), jnp.float32),
            jax.ShapeDtypeStruct((B, N), jnp.float32),
            jax.ShapeDtypeStruct((3, B, N), jnp.float32),
        ),
    )(xyz)


GB = 4
NGB = S_PER_TILE // GB


def _sc_group_body(xyzb_hbm, xx_hbm, nxyzb_hbm, cc_hbm, f_hbm, out_hbm,
                   xv, yv, zv, xxv, cxv, cyv, czv, ccv, idxbuf, idxsel, gbuf,
                   gsem0, gsem1, osem0, osem1):
    cid = lax.axis_index("c")
    sid = lax.axis_index("s")
    wid = sid * 2 + cid
    b = wid // (NTILES // B)
    q = wid % (NTILES // B)
    s0 = q * S_PER_TILE

    pltpu.sync_copy(xyzb_hbm.at[pl.ds((0 * B + b) * N, N)], xv)
    pltpu.sync_copy(xyzb_hbm.at[pl.ds((1 * B + b) * N, N)], yv)
    pltpu.sync_copy(xyzb_hbm.at[pl.ds((2 * B + b) * N, N)], zv)
    pltpu.sync_copy(xx_hbm.at[pl.ds(b * N, N)], xxv)
    pltpu.sync_copy(nxyzb_hbm.at[pl.ds((0 * B + b) * S + s0, S_PER_TILE)], cxv)
    pltpu.sync_copy(nxyzb_hbm.at[pl.ds((1 * B + b) * S + s0, S_PER_TILE)], cyv)
    pltpu.sync_copy(nxyzb_hbm.at[pl.ds((2 * B + b) * S + s0, S_PER_TILE)], czv)
    pltpu.sync_copy(cc_hbm.at[pl.ds(b * S + s0, S_PER_TILE)], ccv)

    lane = lax.iota(jnp.int32, 16)
    gsems = (gsem0, gsem1)
    osems = (osem0, osem1)

    def scan_batch(k, p):
        for j in range(GB):
            ci = k * GB + j
            ii = jnp.full((16,), ci, jnp.int32)
            cxs = plsc.load_gather(cxv, [ii])
            cys = plsc.load_gather(cyv, [ii])
            czs = plsc.load_gather(czv, [ii])
            ccs = plsc.load_gather(ccv, [ii])

            def cond(st):
                jj, cnt = st
                return (jj < NCHUNK) & (cnt < NS)

            def chunk(st):
                jj, cnt = st
                off = jj * 16
                xc = xv[pl.ds(off, 16)]
                yc = yv[pl.ds(off, 16)]
                zc = zv[pl.ds(off, 16)]
                xxc = xxv[pl.ds(off, 16)]
                dot = (cxs * xc + cys * yc) + czs * zc
                d = (ccs + xxc) - 2.0 * dot
                m = d <= R2
                plsc.store_compressed(idxbuf.at[pl.ds(cnt, 16)],
                                      lane + off, mask=m)
                return jj + 1, cnt + jnp.sum(m.astype(jnp.int32))

            _, cnt = lax.while_loop(cond, chunk, (jnp.int32(0), jnp.int32(0)))

            first = plsc.load_gather(idxbuf, [jnp.zeros((16,), jnp.int32)])
            for h in range(NS // 16):
                v = idxbuf[pl.ds(h * 16, 16)]
                v = jnp.where(lane + h * 16 < cnt, v, first)
                idxsel[p, pl.ds(j * NS + h * 16, 16)] = v

    def gather_start(p):
        pltpu.async_copy(f_hbm.at[b].at[idxsel.at[p]], gbuf.at[p], gsems[p])

    def gather_wait(p):
        pltpu.make_async_copy(f_hbm.at[b].at[idxsel.at[p]], gbuf.at[p],
                              gsems[p]).wait()

    def out_dst(k):
        return out_hbm.at[b, pl.ds((s0 + k * GB) * NS, GB * NS)]

    def outcopy_start(k, p):
        pltpu.async_copy(gbuf.at[p], out_dst(k), osems[p])

    def outcopy_wait(k, p):
        pltpu.make_async_copy(gbuf.at[p], out_dst(k), osems[p]).wait()

    def pair(t, carry):
        k0 = 2 * t

        @pl.when(t >= 1)
        def _():
            outcopy_wait(k0, 0)
        scan_batch(k0, 0)

        @pl.when(t >= 1)
        def _():
            gather_wait(1)
            outcopy_start(k0 - 1, 1)
        gather_start(0)

        k1 = 2 * t + 1

        @pl.when(t >= 1)
        def _():
            outcopy_wait(k1, 1)
        scan_batch(k1, 1)

        gather_wait(0)
        outcopy_start(k0, 0)
        gather_start(1)
        return carry

    lax.fori_loop(0, NGB // 2, pair, jnp.int32(0))

    k_last = NGB - 1
    gather_wait(1)
    outcopy_start(k_last, 1)
    outcopy_wait(k_last - 1, 0)
    outcopy_wait(k_last, 1)


@functools.partial(
    pl.kernel,
    out_type=jax.ShapeDtypeStruct((B, S * NS, DPAD), jnp.float32),
    mesh=plsc.VectorSubcoreMesh(
        core_axis_name="c", subcore_axis_name="s", num_cores=2, num_subcores=16
    ),
    compiler_params=pltpu.CompilerParams(needs_layout_passes=False),
    scratch_types=[
        pltpu.VMEM((N,), jnp.float32),
        pltpu.VMEM((N,), jnp.float32),
        pltpu.VMEM((N,), jnp.float32),
        pltpu.VMEM((N,), jnp.float32),
        pltpu.VMEM((S_PER_TILE,), jnp.float32),
        pltpu.VMEM((S_PER_TILE,), jnp.float32),
        pltpu.VMEM((S_PER_TILE,), jnp.float32),
        pltpu.VMEM((S_PER_TILE,), jnp.float32),
        pltpu.VMEM((NS + 16,), jnp.int32),
        pltpu.VMEM((2, GB * NS), jnp.int32),
        pltpu.VMEM((2, GB * NS, DPAD), jnp.float32),
        pltpu.SemaphoreType.DMA,
        pltpu.SemaphoreType.DMA,
        pltpu.SemaphoreType.DMA,
        pltpu.SemaphoreType.DMA,
    ],
)
def _sc_group(*refs):
    _sc_group_body(*refs)


SBLK = 128


def _mlp_body(g_ref, c_ref, w1_ref, a1_ref, w2_ref, a2_ref, w3_ref, a3_ref,
              out_ref):
    X = g_ref[0].reshape(SBLK * NS, DPAD)
    h = jnp.dot(X, w1_ref[...], preferred_element_type=jnp.float32)
    h = h + a1_ref[...]
    corr = jnp.dot(c_ref[0], w1_ref[0:3, :], preferred_element_type=jnp.float32)
    h = h.reshape(SBLK, NS, 64) - corr[:, None, :]
    h = jnp.maximum(h, 0.0).reshape(SBLK * NS, 64)
    h = jnp.dot(h, w2_ref[...], preferred_element_type=jnp.float32) + a2_ref[...]
    h = jnp.maximum(h, 0.0)
    h = jnp.dot(h, w3_ref[...], preferred_element_type=jnp.float32) + a3_ref[...]
    h = jnp.maximum(h, 0.0)
    out_ref[0] = jnp.max(h.reshape(SBLK, NS, 128), axis=1)


def _mlp(G, c, w1, a1, w2, a2, w3, a3):
    grid = (B, S // SBLK)
    return pl.pallas_call(
        _mlp_body,
        grid=grid,
        in_specs=[
            pl.BlockSpec((1, SBLK, NS, DPAD), lambda b, s: (b, s, 0, 0)),
            pl.BlockSpec((1, SBLK, 3), lambda b, s: (b, s, 0)),
            pl.BlockSpec((DPAD, 64), lambda b, s: (0, 0)),
            pl.BlockSpec((1, 64), lambda b, s: (0, 0)),
            pl.BlockSpec((64, 64), lambda b, s: (0, 0)),
            pl.BlockSpec((1, 64), lambda b, s: (0, 0)),
            pl.BlockSpec((64, 128), lambda b, s: (0, 0)),
            pl.BlockSpec((1, 128), lambda b, s: (0, 0)),
        ],
        out_specs=pl.BlockSpec((1, SBLK, 128), lambda b, s: (b, s, 0)),
        out_shape=jax.ShapeDtypeStruct((B, S, 128), jnp.float32),
    )(G, c, w1, a1, w2, a2, w3, a3)


def kernel(xyz, points, W1, b1, g1, be1, W2, b2, g2, be2, W3, b3, g3, be3):
    nxyz, nxyzb, cc, xx, xyzb = _fps(xyz)

    xyz_t = jnp.transpose(xyz, (0, 2, 1))
    pts_t = jnp.transpose(points, (0, 2, 1))
    F = jnp.concatenate(
        [xyz_t, pts_t, jnp.zeros((B, N, DPAD - 67), jnp.float32)], axis=-1)

    G = _sc_group(xyzb.reshape(-1), xx.reshape(-1), nxyzb.reshape(-1),
                  cc.reshape(-1), F).reshape(B, S, NS, DPAD)

    w1 = jnp.pad(W1 * g1[:, None], ((0, 0), (0, DPAD - 67))).T
    a1 = (b1 * g1 + be1)[None, :]
    w2 = (W2 * g2[:, None]).T
    a2 = (b2 * g2 + be2)[None, :]
    w3 = (W3 * g3[:, None]).T
    a3 = (b3 * g3 + be3)[None, :]

    c = jnp.transpose(nxyz, (1, 2, 0))
    out = _mlp(G, c, w1, a1, w2, a2, w3, a3)

    new_xyz = jnp.transpose(nxyz, (1, 0, 2))
    new_points = jnp.transpose(out, (0, 2, 1))
    return new_xyz, new_points


# speedup vs baseline: 368.7961x; 1.5129x over previous
import functools

import jax
import jax.numpy as jnp
from jax import lax
from jax.experimental import pallas as pl
from jax.experimental.pallas import tpu as pltpu
from jax.experimental.pallas import tpu_sc as plsc

B = 8
N = 4096
S = 512
NS = 32
R2 = 0.2 ** 2
DPAD = 128
NTILES = 32
S_PER_TILE = S * B // NTILES
NCHUNK = N // 16


def _r(v):
    return v.astype(jnp.bfloat16).astype(jnp.float32)


def _fps_body(xyz_ref, nxyz_ref, nxyzb_ref, cc_ref, xx_ref, xyzb_ref):
    x = xyz_ref[:, 0, :]
    y = xyz_ref[:, 1, :]
    z = xyz_ref[:, 2, :]
    xx_ref[...] = (x * x + y * y) + z * z
    xyzb_ref[0] = _r(x)
    xyzb_ref[1] = _r(y)
    xyzb_ref[2] = _r(z)
    iota = lax.broadcasted_iota(jnp.int32, (B, N), 1)
    iota_s = lax.broadcasted_iota(jnp.int32, (B, S), 1)

    def body(i, st):
        dist, far, nx, ny, nz = st
        sel = iota == far
        cx = jnp.sum(jnp.where(sel, x, 0.0), axis=1, keepdims=True)
        cy = jnp.sum(jnp.where(sel, y, 0.0), axis=1, keepdims=True)
        cz = jnp.sum(jnp.where(sel, z, 0.0), axis=1, keepdims=True)
        sel_s = iota_s == i
        nx = jnp.where(sel_s, cx, nx)
        ny = jnp.where(sel_s, cy, ny)
        nz = jnp.where(sel_s, cz, nz)
        dx, dy, dz = x - cx, y - cy, z - cz
        d = (dx * dx + dy * dy) + dz * dz
        dist = jnp.minimum(dist, d)
        mx = jnp.max(dist, axis=1, keepdims=True)
        far = jnp.min(jnp.where(dist == mx, iota, N), axis=1, keepdims=True)
        return dist, far, nx, ny, nz

    dist0 = jnp.full((B, N), 1e10, jnp.float32)
    far0 = jnp.zeros((B, 1), jnp.int32)
    z0 = jnp.zeros((B, S), jnp.float32)
    _, _, nx, ny, nz = lax.fori_loop(0, S, body, (dist0, far0, z0, z0, z0))
    nxyz_ref[0] = nx
    nxyz_ref[1] = ny
    nxyz_ref[2] = nz
    nxyzb_ref[0] = _r(nx)
    nxyzb_ref[1] = _r(ny)
    nxyzb_ref[2] = _r(nz)
    cc_ref[...] = (nx * nx + ny * ny) + nz * nz


def _fps(xyz):
    return pl.pallas_call(
        _fps_body,
        out_shape=(
            jax.ShapeDtypeStruct((3, B, S), jnp.float32),
            jax.ShapeDtypeStruct((3, B, S), jnp.float32),
            jax.ShapeDtypeStruct((B, S), jnp.float32),
            jax.ShapeDtypeStruct((B, N), jnp.float32),
            jax.ShapeDtypeStruct((3, B, N), jnp.float32),
        ),
    )(xyz)


GB = 4
NGB = S_PER_TILE // GB
UNROLL = 4


def _sc_group_body(xyzb_hbm, xx_hbm, nxyzb_hbm, cc_hbm, f_hbm, out_hbm,
                   xv, yv, zv, xxv, cxv, cyv, czv, ccv, idxbuf, idxsel, gbuf,
                   gsem0, gsem1, osem0, osem1):
    cid = lax.axis_index("c")
    sid = lax.axis_index("s")
    wid = sid * 2 + cid
    b = wid // (NTILES // B)
    q = wid % (NTILES // B)
    s0 = q * S_PER_TILE

    pltpu.sync_copy(xyzb_hbm.at[pl.ds((0 * B + b) * N, N)], xv)
    pltpu.sync_copy(xyzb_hbm.at[pl.ds((1 * B + b) * N, N)], yv)
    pltpu.sync_copy(xyzb_hbm.at[pl.ds((2 * B + b) * N, N)], zv)
    pltpu.sync_copy(xx_hbm.at[pl.ds(b * N, N)], xxv)
    pltpu.sync_copy(nxyzb_hbm.at[pl.ds((0 * B + b) * S + s0, S_PER_TILE)], cxv)
    pltpu.sync_copy(nxyzb_hbm.at[pl.ds((1 * B + b) * S + s0, S_PER_TILE)], cyv)
    pltpu.sync_copy(nxyzb_hbm.at[pl.ds((2 * B + b) * S + s0, S_PER_TILE)], czv)
    pltpu.sync_copy(cc_hbm.at[pl.ds(b * S + s0, S_PER_TILE)], ccv)

    lane = lax.iota(jnp.int32, 16)
    gsems = (gsem0, gsem1)
    osems = (osem0, osem1)

    def scan_batch(k, p):
        for j in range(GB):
            ci = k * GB + j
            ii = jnp.full((16,), ci, jnp.int32)
            cxs = plsc.load_gather(cxv, [ii])
            cys = plsc.load_gather(cyv, [ii])
            czs = plsc.load_gather(czv, [ii])
            ccs = plsc.load_gather(ccv, [ii])

            def cond(st):
                jj, cnt = st
                return (jj < NCHUNK // UNROLL) & (cnt < NS)

            def chunk(st):
                jj, cnt = st
                cntv = jnp.broadcast_to(cnt, (16,))
                masks = []
                for u in range(UNROLL):
                    off = (jj * UNROLL + u) * 16
                    xc = xv[pl.ds(off, 16)]
                    yc = yv[pl.ds(off, 16)]
                    zc = zv[pl.ds(off, 16)]
                    xxc = xxv[pl.ds(off, 16)]
                    dot = (cxs * xc + cys * yc) + czs * zc
                    d = (ccs + xxc) - 2.0 * dot
                    masks.append(d <= R2)
                offv = cntv
                for u in range(UNROLL):
                    m = masks[u]
                    off = (jj * UNROLL + u) * 16
                    pos = plsc.cumsum(m.astype(jnp.int32)) + offv
                    plsc.store_scatter(idxbuf, [pos - 1], lane + off, mask=m)
                    if u + 1 < UNROLL:
                        offv = offv + plsc.all_reduce_population_count(m)
                total = masks[0].astype(jnp.int32)
                for u in range(1, UNROLL):
                    total = total + masks[u].astype(jnp.int32)
                return jj + 1, cnt + jnp.sum(total)

            _, cnt = lax.while_loop(cond, chunk, (jnp.int32(0), jnp.int32(0)))

            first = plsc.load_gather(idxbuf, [jnp.zeros((16,), jnp.int32)])
            for h in range(NS // 16):
                v = idxbuf[pl.ds(h * 16, 16)]
                v = jnp.where(lane + h * 16 < cnt, v, first)
                idxsel[p, pl.ds(j * NS + h * 16, 16)] = v

    def gather_start(p):
        pltpu.async_copy(f_hbm.at[b].at[idxsel.at[p]], gbuf.at[p], gsems[p])

    def gather_wait(p):
        pltpu.make_async_copy(f_hbm.at[b].at[idxsel.at[p]], gbuf.at[p],
                              gsems[p]).wait()

    def out_dst(k):
        return out_hbm.at[b, pl.ds((s0 + k * GB) * NS, GB * NS)]

    def outcopy_start(k, p):
        pltpu.async_copy(gbuf.at[p], out_dst(k), osems[p])

    def outcopy_wait(k, p):
        pltpu.make_async_copy(gbuf.at[p], out_dst(k), osems[p]).wait()

    def pair(t, carry):
        k0 = 2 * t

        @pl.when(t >= 1)
        def _():
            outcopy_wait(k0, 0)
        scan_batch(k0, 0)

        @pl.when(t >= 1)
        def _():
            gather_wait(1)
            outcopy_start(k0 - 1, 1)
        gather_start(0)

        k1 = 2 * t + 1

        @pl.when(t >= 1)
        def _():
            outcopy_wait(k1, 1)
        scan_batch(k1, 1)

        gather_wait(0)
        outcopy_start(k0, 0)
        gather_start(1)
        return carry

    lax.fori_loop(0, NGB // 2, pair, jnp.int32(0))

    k_last = NGB - 1
    gather_wait(1)
    outcopy_start(k_last, 1)
    outcopy_wait(k_last - 1, 0)
    outcopy_wait(k_last, 1)


@functools.partial(
    pl.kernel,
    out_type=jax.ShapeDtypeStruct((B, S * NS, DPAD), jnp.float32),
    mesh=plsc.VectorSubcoreMesh(
        core_axis_name="c", subcore_axis_name="s", num_cores=2, num_subcores=16
    ),
    compiler_params=pltpu.CompilerParams(needs_layout_passes=False),
    scratch_types=[
        pltpu.VMEM((N,), jnp.float32),
        pltpu.VMEM((N,), jnp.float32),
        pltpu.VMEM((N,), jnp.float32),
        pltpu.VMEM((N,), jnp.float32),
        pltpu.VMEM((S_PER_TILE,), jnp.float32),
        pltpu.VMEM((S_PER_TILE,), jnp.float32),
        pltpu.VMEM((S_PER_TILE,), jnp.float32),
        pltpu.VMEM((S_PER_TILE,), jnp.float32),
        pltpu.VMEM((NS + 16 * UNROLL,), jnp.int32),
        pltpu.VMEM((2, GB * NS), jnp.int32),
        pltpu.VMEM((2, GB * NS, DPAD), jnp.float32),
        pltpu.SemaphoreType.DMA,
        pltpu.SemaphoreType.DMA,
        pltpu.SemaphoreType.DMA,
        pltpu.SemaphoreType.DMA,
    ],
)
def _sc_group(*refs):
    _sc_group_body(*refs)


SBLK = 128


def _mlp_body(g_ref, c_ref, w1_ref, a1_ref, w2_ref, a2_ref, w3_ref, a3_ref,
              out_ref):
    X = g_ref[0].reshape(SBLK * NS, DPAD)
    h = jnp.dot(X, w1_ref[...], preferred_element_type=jnp.float32)
    h = h + a1_ref[...]
    corr = jnp.dot(c_ref[0], w1_ref[0:3, :], preferred_element_type=jnp.float32)
    h = h.reshape(SBLK, NS, 64) - corr[:, None, :]
    h = jnp.maximum(h, 0.0).reshape(SBLK * NS, 64)
    h = jnp.dot(h, w2_ref[...], preferred_element_type=jnp.float32) + a2_ref[...]
    h = jnp.maximum(h, 0.0)
    h = jnp.dot(h, w3_ref[...], preferred_element_type=jnp.float32) + a3_ref[...]
    h = jnp.maximum(h, 0.0)
    out_ref[0] = jnp.max(h.reshape(SBLK, NS, 128), axis=1)


def _mlp(G, c, w1, a1, w2, a2, w3, a3):
    grid = (B, S // SBLK)
    return pl.pallas_call(
        _mlp_body,
        grid=grid,
        in_specs=[
            pl.BlockSpec((1, SBLK, NS, DPAD), lambda b, s: (b, s, 0, 0)),
            pl.BlockSpec((1, SBLK, 3), lambda b, s: (b, s, 0)),
            pl.BlockSpec((DPAD, 64), lambda b, s: (0, 0)),
            pl.BlockSpec((1, 64), lambda b, s: (0, 0)),
            pl.BlockSpec((64, 64), lambda b, s: (0, 0)),
            pl.BlockSpec((1, 64), lambda b, s: (0, 0)),
            pl.BlockSpec((64, 128), lambda b, s: (0, 0)),
            pl.BlockSpec((1, 128), lambda b, s: (0, 0)),
        ],
        out_specs=pl.BlockSpec((1, SBLK, 128), lambda b, s: (b, s, 0)),
        out_shape=jax.ShapeDtypeStruct((B, S, 128), jnp.float32),
    )(G, c, w1, a1, w2, a2, w3, a3)


def kernel(xyz, points, W1, b1, g1, be1, W2, b2, g2, be2, W3, b3, g3, be3):
    nxyz, nxyzb, cc, xx, xyzb = _fps(xyz)

    xyz_t = jnp.transpose(xyz, (0, 2, 1))
    pts_t = jnp.transpose(points, (0, 2, 1))
    F = jnp.concatenate(
        [xyz_t, pts_t, jnp.zeros((B, N, DPAD - 67), jnp.float32)], axis=-1)

    G = _sc_group(xyzb.reshape(-1), xx.reshape(-1), nxyzb.reshape(-1),
                  cc.reshape(-1), F).reshape(B, S, NS, DPAD)

    w1 = jnp.pad(W1 * g1[:, None], ((0, 0), (0, DPAD - 67))).T
    a1 = (b1 * g1 + be1)[None, :]
    w2 = (W2 * g2[:, None]).T
    a2 = (b2 * g2 + be2)[None, :]
    w3 = (W3 * g3[:, None]).T
    a3 = (b3 * g3 + be3)[None, :]

    c = jnp.transpose(nxyz, (1, 2, 0))
    out = _mlp(G, c, w1, a1, w2, a2, w3, a3)

    new_xyz = jnp.transpose(nxyz, (1, 0, 2))
    new_points = jnp.transpose(out, (0, 2, 1))
    return new_xyz, new_points


# speedup vs baseline: 399.6756x; 1.0837x over previous
import functools

import jax
import jax.numpy as jnp
from jax import lax
from jax.experimental import pallas as pl
from jax.experimental.pallas import tpu as pltpu
from jax.experimental.pallas import tpu_sc as plsc

B = 8
N = 4096
S = 512
NS = 32
R2 = 0.2 ** 2
DPAD = 128
NTILES = 32
S_PER_TILE = S * B // NTILES
NCHUNK = N // 16


def _r(v):
    return v.astype(jnp.bfloat16).astype(jnp.float32)


def _fps_body(xyz_ref, nxyz_ref, nxyzb_ref, cc_ref, xx_ref, xyzb_ref):
    x = xyz_ref[:, 0, :]
    y = xyz_ref[:, 1, :]
    z = xyz_ref[:, 2, :]
    xx_ref[...] = (x * x + y * y) + z * z
    xyzb_ref[0] = _r(x)
    xyzb_ref[1] = _r(y)
    xyzb_ref[2] = _r(z)
    iota = lax.broadcasted_iota(jnp.int32, (B, N), 1)
    iota_s = lax.broadcasted_iota(jnp.int32, (B, S), 1)

    def body(i, st):
        dist, far, nx, ny, nz = st
        sel = iota == far
        cx = jnp.sum(jnp.where(sel, x, 0.0), axis=1, keepdims=True)
        cy = jnp.sum(jnp.where(sel, y, 0.0), axis=1, keepdims=True)
        cz = jnp.sum(jnp.where(sel, z, 0.0), axis=1, keepdims=True)
        sel_s = iota_s == i
        nx = jnp.where(sel_s, cx, nx)
        ny = jnp.where(sel_s, cy, ny)
        nz = jnp.where(sel_s, cz, nz)
        dx, dy, dz = x - cx, y - cy, z - cz
        d = (dx * dx + dy * dy) + dz * dz
        dist = jnp.minimum(dist, d)
        mx = jnp.max(dist, axis=1, keepdims=True)
        far = jnp.min(jnp.where(dist == mx, iota, N), axis=1, keepdims=True)
        return dist, far, nx, ny, nz

    dist0 = jnp.full((B, N), 1e10, jnp.float32)
    far0 = jnp.zeros((B, 1), jnp.int32)
    z0 = jnp.zeros((B, S), jnp.float32)
    _, _, nx, ny, nz = lax.fori_loop(0, S, body, (dist0, far0, z0, z0, z0))
    nxyz_ref[0] = nx
    nxyz_ref[1] = ny
    nxyz_ref[2] = nz
    nxyzb_ref[0] = _r(nx)
    nxyzb_ref[1] = _r(ny)
    nxyzb_ref[2] = _r(nz)
    cc_ref[...] = (nx * nx + ny * ny) + nz * nz


def _fps(xyz):
    return pl.pallas_call(
        _fps_body,
        out_shape=(
            jax.ShapeDtypeStruct((3, B, S), jnp.float32),
            jax.ShapeDtypeStruct((3, B, S), jnp.float32),
            jax.ShapeDtypeStruct((B, S), jnp.float32),
            jax.ShapeDtypeStruct((B, N), jnp.float32),
            jax.ShapeDtypeStruct((3, B, N), jnp.float32),
        ),
    )(xyz)


GB = 4
NGB = S_PER_TILE // GB
UNROLL = 8


def _sc_group_body(xyzb_hbm, xx_hbm, nxyzb_hbm, cc_hbm, f_hbm, out_hbm,
                   xv, yv, zv, xxv, cxv, cyv, czv, ccv, idxbuf, idxsel, gbuf,
                   gsem0, gsem1, osem0, osem1):
    cid = lax.axis_index("c")
    sid = lax.axis_index("s")
    wid = sid * 2 + cid
    b = wid // (NTILES // B)
    q = wid % (NTILES // B)
    s0 = q * S_PER_TILE

    pltpu.sync_copy(xyzb_hbm.at[pl.ds((0 * B + b) * N, N)], xv)
    pltpu.sync_copy(xyzb_hbm.at[pl.ds((1 * B + b) * N, N)], yv)
    pltpu.sync_copy(xyzb_hbm.at[pl.ds((2 * B + b) * N, N)], zv)
    pltpu.sync_copy(xx_hbm.at[pl.ds(b * N, N)], xxv)
    pltpu.sync_copy(nxyzb_hbm.at[pl.ds((0 * B + b) * S + s0, S_PER_TILE)], cxv)
    pltpu.sync_copy(nxyzb_hbm.at[pl.ds((1 * B + b) * S + s0, S_PER_TILE)], cyv)
    pltpu.sync_copy(nxyzb_hbm.at[pl.ds((2 * B + b) * S + s0, S_PER_TILE)], czv)
    pltpu.sync_copy(cc_hbm.at[pl.ds(b * S + s0, S_PER_TILE)], ccv)

    lane = lax.iota(jnp.int32, 16)
    gsems = (gsem0, gsem1)
    osems = (osem0, osem1)

    def scan_batch(k, p):
        for j in range(GB):
            ci = k * GB + j
            ii = jnp.full((16,), ci, jnp.int32)
            cxs = plsc.load_gather(cxv, [ii])
            cys = plsc.load_gather(cyv, [ii])
            czs = plsc.load_gather(czv, [ii])
            ccs = plsc.load_gather(ccv, [ii])

            def cond(st):
                jj, cnt = st
                return (jj < NCHUNK // UNROLL) & (cnt < NS)

            def chunk(st):
                jj, cnt = st
                cntv = jnp.broadcast_to(cnt, (16,))
                masks = []
                for u in range(UNROLL):
                    off = (jj * UNROLL + u) * 16
                    xc = xv[pl.ds(off, 16)]
                    yc = yv[pl.ds(off, 16)]
                    zc = zv[pl.ds(off, 16)]
                    xxc = xxv[pl.ds(off, 16)]
                    dot = (cxs * xc + cys * yc) + czs * zc
                    d = (ccs + xxc) - 2.0 * dot
                    masks.append(d <= R2)
                offv = cntv
                for u in range(UNROLL):
                    m = masks[u]
                    off = (jj * UNROLL + u) * 16
                    pos = plsc.cumsum(m.astype(jnp.int32)) + offv
                    plsc.store_scatter(idxbuf, [pos - 1], lane + off, mask=m)
                    if u + 1 < UNROLL:
                        offv = offv + plsc.all_reduce_population_count(m)
                total = masks[0].astype(jnp.int32)
                for u in range(1, UNROLL):
                    total = total + masks[u].astype(jnp.int32)
                return jj + 1, cnt + jnp.sum(total)

            _, cnt = lax.while_loop(cond, chunk, (jnp.int32(0), jnp.int32(0)))

            first = plsc.load_gather(idxbuf, [jnp.zeros((16,), jnp.int32)])
            for h in range(NS // 16):
                v = idxbuf[pl.ds(h * 16, 16)]
                v = jnp.where(lane + h * 16 < cnt, v, first)
                idxsel[p, pl.ds(j * NS + h * 16, 16)] = v

    def gather_start(p):
        pltpu.async_copy(f_hbm.at[b].at[idxsel.at[p]], gbuf.at[p], gsems[p])

    def gather_wait(p):
        pltpu.make_async_copy(f_hbm.at[b].at[idxsel.at[p]], gbuf.at[p],
                              gsems[p]).wait()

    def out_dst(k):
        return out_hbm.at[b, pl.ds((s0 + k * GB) * NS, GB * NS)]

    def outcopy_start(k, p):
        pltpu.async_copy(gbuf.at[p], out_dst(k), osems[p])

    def outcopy_wait(k, p):
        pltpu.make_async_copy(gbuf.at[p], out_dst(k), osems[p]).wait()

    def pair(t, carry):
        k0 = 2 * t

        @pl.when(t >= 1)
        def _():
            outcopy_wait(k0, 0)
        scan_batch(k0, 0)

        @pl.when(t >= 1)
        def _():
            gather_wait(1)
            outcopy_start(k0 - 1, 1)
        gather_start(0)

        k1 = 2 * t + 1

        @pl.when(t >= 1)
        def _():
            outcopy_wait(k1, 1)
        scan_batch(k1, 1)

        gather_wait(0)
        outcopy_start(k0, 0)
        gather_start(1)
        return carry

    lax.fori_loop(0, NGB // 2, pair, jnp.int32(0))

    k_last = NGB - 1
    gather_wait(1)
    outcopy_start(k_last, 1)
    outcopy_wait(k_last - 1, 0)
    outcopy_wait(k_last, 1)


@functools.partial(
    pl.kernel,
    out_type=jax.ShapeDtypeStruct((B, S * NS, DPAD), jnp.float32),
    mesh=plsc.VectorSubcoreMesh(
        core_axis_name="c", subcore_axis_name="s", num_cores=2, num_subcores=16
    ),
    compiler_params=pltpu.CompilerParams(needs_layout_passes=False),
    scratch_types=[
        pltpu.VMEM((N,), jnp.float32),
        pltpu.VMEM((N,), jnp.float32),
        pltpu.VMEM((N,), jnp.float32),
        pltpu.VMEM((N,), jnp.float32),
        pltpu.VMEM((S_PER_TILE,), jnp.float32),
        pltpu.VMEM((S_PER_TILE,), jnp.float32),
        pltpu.VMEM((S_PER_TILE,), jnp.float32),
        pltpu.VMEM((S_PER_TILE,), jnp.float32),
        pltpu.VMEM((NS + 16 * UNROLL,), jnp.int32),
        pltpu.VMEM((2, GB * NS), jnp.int32),
        pltpu.VMEM((2, GB * NS, DPAD), jnp.float32),
        pltpu.SemaphoreType.DMA,
        pltpu.SemaphoreType.DMA,
        pltpu.SemaphoreType.DMA,
        pltpu.SemaphoreType.DMA,
    ],
)
def _sc_group(*refs):
    _sc_group_body(*refs)


SBLK = 128


def _mlp_body(g_ref, c_ref, w1_ref, a1_ref, w2_ref, a2_ref, w3_ref, a3_ref,
              out_ref):
    X = g_ref[0].reshape(SBLK * NS, DPAD)
    h = jnp.dot(X, w1_ref[...], preferred_element_type=jnp.float32)
    h = h + a1_ref[...]
    corr = jnp.dot(c_ref[0], w1_ref[0:3, :], preferred_element_type=jnp.float32)
    h = h.reshape(SBLK, NS, 64) - corr[:, None, :]
    h = jnp.maximum(h, 0.0).reshape(SBLK * NS, 64)
    h = jnp.dot(h, w2_ref[...], preferred_element_type=jnp.float32) + a2_ref[...]
    h = jnp.maximum(h, 0.0)
    h = jnp.dot(h, w3_ref[...], preferred_element_type=jnp.float32) + a3_ref[...]
    h = jnp.maximum(h, 0.0)
    out_ref[0] = jnp.max(h.reshape(SBLK, NS, 128), axis=1)


def _mlp(G, c, w1, a1, w2, a2, w3, a3):
    grid = (B, S // SBLK)
    return pl.pallas_call(
        _mlp_body,
        grid=grid,
        in_specs=[
            pl.BlockSpec((1, SBLK, NS, DPAD), lambda b, s: (b, s, 0, 0)),
            pl.BlockSpec((1, SBLK, 3), lambda b, s: (b, s, 0)),
            pl.BlockSpec((DPAD, 64), lambda b, s: (0, 0)),
            pl.BlockSpec((1, 64), lambda b, s: (0, 0)),
            pl.BlockSpec((64, 64), lambda b, s: (0, 0)),
            pl.BlockSpec((1, 64), lambda b, s: (0, 0)),
            pl.BlockSpec((64, 128), lambda b, s: (0, 0)),
            pl.BlockSpec((1, 128), lambda b, s: (0, 0)),
        ],
        out_specs=pl.BlockSpec((1, SBLK, 128), lambda b, s: (b, s, 0)),
        out_shape=jax.ShapeDtypeStruct((B, S, 128), jnp.float32),
    )(G, c, w1, a1, w2, a2, w3, a3)


def kernel(xyz, points, W1, b1, g1, be1, W2, b2, g2, be2, W3, b3, g3, be3):
    nxyz, nxyzb, cc, xx, xyzb = _fps(xyz)

    xyz_t = jnp.transpose(xyz, (0, 2, 1))
    pts_t = jnp.transpose(points, (0, 2, 1))
    F = jnp.concatenate(
        [xyz_t, pts_t, jnp.zeros((B, N, DPAD - 67), jnp.float32)], axis=-1)

    G = _sc_group(xyzb.reshape(-1), xx.reshape(-1), nxyzb.reshape(-1),
                  cc.reshape(-1), F).reshape(B, S, NS, DPAD)

    w1 = jnp.pad(W1 * g1[:, None], ((0, 0), (0, DPAD - 67))).T
    a1 = (b1 * g1 + be1)[None, :]
    w2 = (W2 * g2[:, None]).T
    a2 = (b2 * g2 + be2)[None, :]
    w3 = (W3 * g3[:, None]).T
    a3 = (b3 * g3 + be3)[None, :]

    c = jnp.transpose(nxyz, (1, 2, 0))
    out = _mlp(G, c, w1, a1, w2, a2, w3, a3)

    new_xyz = jnp.transpose(nxyz, (1, 0, 2))
    new_points = jnp.transpose(out, (0, 2, 1))
    return new_xyz, new_points


# speedup vs baseline: 416.7097x; 1.0426x over previous
import functools

import jax
import jax.numpy as jnp
from jax import lax
from jax.experimental import pallas as pl
from jax.experimental.pallas import tpu as pltpu
from jax.experimental.pallas import tpu_sc as plsc

B = 8
N = 4096
S = 512
NS = 32
R2 = 0.2 ** 2
DPAD = 128
NTILES = 32
S_PER_TILE = S * B // NTILES
NCHUNK = N // 16


def _r(v):
    return v.astype(jnp.bfloat16).astype(jnp.float32)


def _fps_body(xyz_ref, nxyz_ref, nxyzb_ref, cc_ref, xx_ref, xyzb_ref):
    x = xyz_ref[:, 0, :]
    y = xyz_ref[:, 1, :]
    z = xyz_ref[:, 2, :]
    xx_ref[...] = (x * x + y * y) + z * z
    xyzb_ref[0] = _r(x)
    xyzb_ref[1] = _r(y)
    xyzb_ref[2] = _r(z)
    iota = lax.broadcasted_iota(jnp.int32, (B, N), 1)
    iota_s = lax.broadcasted_iota(jnp.int32, (B, S), 1)

    def body(i, st):
        dist, far, nx, ny, nz = st
        sel = iota == far
        cx = jnp.sum(jnp.where(sel, x, 0.0), axis=1, keepdims=True)
        cy = jnp.sum(jnp.where(sel, y, 0.0), axis=1, keepdims=True)
        cz = jnp.sum(jnp.where(sel, z, 0.0), axis=1, keepdims=True)
        sel_s = iota_s == i
        nx = jnp.where(sel_s, cx, nx)
        ny = jnp.where(sel_s, cy, ny)
        nz = jnp.where(sel_s, cz, nz)
        dx, dy, dz = x - cx, y - cy, z - cz
        d = (dx * dx + dy * dy) + dz * dz
        dist = jnp.minimum(dist, d)
        mx = jnp.max(dist, axis=1, keepdims=True)
        far = jnp.min(jnp.where(dist == mx, iota, N), axis=1, keepdims=True)
        return dist, far, nx, ny, nz

    dist0 = jnp.full((B, N), 1e10, jnp.float32)
    far0 = jnp.zeros((B, 1), jnp.int32)
    z0 = jnp.zeros((B, S), jnp.float32)
    _, _, nx, ny, nz = lax.fori_loop(0, S, body, (dist0, far0, z0, z0, z0))
    nxyz_ref[0] = nx
    nxyz_ref[1] = ny
    nxyz_ref[2] = nz
    nxyzb_ref[0] = _r(nx)
    nxyzb_ref[1] = _r(ny)
    nxyzb_ref[2] = _r(nz)
    cc_ref[...] = (nx * nx + ny * ny) + nz * nz


def _fps(xyz):
    return pl.pallas_call(
        _fps_body,
        out_shape=(
            jax.ShapeDtypeStruct((3, B, S), jnp.float32),
            jax.ShapeDtypeStruct((3, B, S), jnp.float32),
            jax.ShapeDtypeStruct((B, S), jnp.float32),
            jax.ShapeDtypeStruct((B, N), jnp.float32),
            jax.ShapeDtypeStruct((3, B, N), jnp.float32),
        ),
    )(xyz)


GB = 4
NGB = S_PER_TILE // GB
UNROLL = 16


def _sc_group_body(xyzb_hbm, xx_hbm, nxyzb_hbm, cc_hbm, f_hbm, out_hbm,
                   xv, yv, zv, xxv, cxv, cyv, czv, ccv, idxbuf, idxsel, gbuf,
                   gsem0, gsem1, osem0, osem1):
    cid = lax.axis_index("c")
    sid = lax.axis_index("s")
    wid = sid * 2 + cid
    b = wid // (NTILES // B)
    q = wid % (NTILES // B)
    s0 = q * S_PER_TILE

    pltpu.sync_copy(xyzb_hbm.at[pl.ds((0 * B + b) * N, N)], xv)
    pltpu.sync_copy(xyzb_hbm.at[pl.ds((1 * B + b) * N, N)], yv)
    pltpu.sync_copy(xyzb_hbm.at[pl.ds((2 * B + b) * N, N)], zv)
    pltpu.sync_copy(xx_hbm.at[pl.ds(b * N, N)], xxv)
    pltpu.sync_copy(nxyzb_hbm.at[pl.ds((0 * B + b) * S + s0, S_PER_TILE)], cxv)
    pltpu.sync_copy(nxyzb_hbm.at[pl.ds((1 * B + b) * S + s0, S_PER_TILE)], cyv)
    pltpu.sync_copy(nxyzb_hbm.at[pl.ds((2 * B + b) * S + s0, S_PER_TILE)], czv)
    pltpu.sync_copy(cc_hbm.at[pl.ds(b * S + s0, S_PER_TILE)], ccv)

    lane = lax.iota(jnp.int32, 16)
    gsems = (gsem0, gsem1)
    osems = (osem0, osem1)

    def scan_batch(k, p):
        for j in range(GB):
            ci = k * GB + j
            ii = jnp.full((16,), ci, jnp.int32)
            cxs = plsc.load_gather(cxv, [ii])
            cys = plsc.load_gather(cyv, [ii])
            czs = plsc.load_gather(czv, [ii])
            ccs = plsc.load_gather(ccv, [ii])

            def cond(st):
                jj, cnt = st
                return (jj < NCHUNK // UNROLL) & (cnt < NS)

            def chunk(st):
                jj, cnt = st
                cntv = jnp.broadcast_to(cnt, (16,))
                masks = []
                for u in range(UNROLL):
                    off = (jj * UNROLL + u) * 16
                    xc = xv[pl.ds(off, 16)]
                    yc = yv[pl.ds(off, 16)]
                    zc = zv[pl.ds(off, 16)]
                    xxc = xxv[pl.ds(off, 16)]
                    dot = (cxs * xc + cys * yc) + czs * zc
                    d = (ccs + xxc) - 2.0 * dot
                    masks.append(d <= R2)
                offv = cntv
                for u in range(UNROLL):
                    m = masks[u]
                    off = (jj * UNROLL + u) * 16
                    pos = plsc.cumsum(m.astype(jnp.int32)) + offv
                    plsc.store_scatter(idxbuf, [pos - 1], lane + off, mask=m)
                    if u + 1 < UNROLL:
                        offv = offv + plsc.all_reduce_population_count(m)
                total = masks[0].astype(jnp.int32)
                for u in range(1, UNROLL):
                    total = total + masks[u].astype(jnp.int32)
                return jj + 1, cnt + jnp.sum(total)

            _, cnt = lax.while_loop(cond, chunk, (jnp.int32(0), jnp.int32(0)))

            first = plsc.load_gather(idxbuf, [jnp.zeros((16,), jnp.int32)])
            for h in range(NS // 16):
                v = idxbuf[pl.ds(h * 16, 16)]
                v = jnp.where(lane + h * 16 < cnt, v, first)
                idxsel[p, pl.ds(j * NS + h * 16, 16)] = v

    def gather_start(p):
        pltpu.async_copy(f_hbm.at[b].at[idxsel.at[p]], gbuf.at[p], gsems[p])

    def gather_wait(p):
        pltpu.make_async_copy(f_hbm.at[b].at[idxsel.at[p]], gbuf.at[p],
                              gsems[p]).wait()

    def out_dst(k):
        return out_hbm.at[b, pl.ds((s0 + k * GB) * NS, GB * NS)]

    def outcopy_start(k, p):
        pltpu.async_copy(gbuf.at[p], out_dst(k), osems[p])

    def outcopy_wait(k, p):
        pltpu.make_async_copy(gbuf.at[p], out_dst(k), osems[p]).wait()

    def pair(t, carry):
        k0 = 2 * t

        @pl.when(t >= 1)
        def _():
            outcopy_wait(k0, 0)
        scan_batch(k0, 0)

        @pl.when(t >= 1)
        def _():
            gather_wait(1)
            outcopy_start(k0 - 1, 1)
        gather_start(0)

        k1 = 2 * t + 1

        @pl.when(t >= 1)
        def _():
            outcopy_wait(k1, 1)
        scan_batch(k1, 1)

        gather_wait(0)
        outcopy_start(k0, 0)
        gather_start(1)
        return carry

    lax.fori_loop(0, NGB // 2, pair, jnp.int32(0))

    k_last = NGB - 1
    gather_wait(1)
    outcopy_start(k_last, 1)
    outcopy_wait(k_last - 1, 0)
    outcopy_wait(k_last, 1)


@functools.partial(
    pl.kernel,
    out_type=jax.ShapeDtypeStruct((B, S * NS, DPAD), jnp.float32),
    mesh=plsc.VectorSubcoreMesh(
        core_axis_name="c", subcore_axis_name="s", num_cores=2, num_subcores=16
    ),
    compiler_params=pltpu.CompilerParams(needs_layout_passes=False),
    scratch_types=[
        pltpu.VMEM((N,), jnp.float32),
        pltpu.VMEM((N,), jnp.float32),
        pltpu.VMEM((N,), jnp.float32),
        pltpu.VMEM((N,), jnp.float32),
        pltpu.VMEM((S_PER_TILE,), jnp.float32),
        pltpu.VMEM((S_PER_TILE,), jnp.float32),
        pltpu.VMEM((S_PER_TILE,), jnp.float32),
        pltpu.VMEM((S_PER_TILE,), jnp.float32),
        pltpu.VMEM((NS + 16 * UNROLL,), jnp.int32),
        pltpu.VMEM((2, GB * NS), jnp.int32),
        pltpu.VMEM((2, GB * NS, DPAD), jnp.float32),
        pltpu.SemaphoreType.DMA,
        pltpu.SemaphoreType.DMA,
        pltpu.SemaphoreType.DMA,
        pltpu.SemaphoreType.DMA,
    ],
)
def _sc_group(*refs):
    _sc_group_body(*refs)


SBLK = 128


def _mlp_body(g_ref, c_ref, w1_ref, a1_ref, w2_ref, a2_ref, w3_ref, a3_ref,
              out_ref):
    X = g_ref[0].reshape(SBLK * NS, DPAD)
    h = jnp.dot(X, w1_ref[...], preferred_element_type=jnp.float32)
    h = h + a1_ref[...]
    corr = jnp.dot(c_ref[0], w1_ref[0:3, :], preferred_element_type=jnp.float32)
    h = h.reshape(SBLK, NS, 64) - corr[:, None, :]
    h = jnp.maximum(h, 0.0).reshape(SBLK * NS, 64)
    h = jnp.dot(h, w2_ref[...], preferred_element_type=jnp.float32) + a2_ref[...]
    h = jnp.maximum(h, 0.0)
    h = jnp.dot(h, w3_ref[...], preferred_element_type=jnp.float32) + a3_ref[...]
    h = jnp.maximum(h, 0.0)
    out_ref[0] = jnp.max(h.reshape(SBLK, NS, 128), axis=1)


def _mlp(G, c, w1, a1, w2, a2, w3, a3):
    grid = (B, S // SBLK)
    return pl.pallas_call(
        _mlp_body,
        grid=grid,
        in_specs=[
            pl.BlockSpec((1, SBLK, NS, DPAD), lambda b, s: (b, s, 0, 0)),
            pl.BlockSpec((1, SBLK, 3), lambda b, s: (b, s, 0)),
            pl.BlockSpec((DPAD, 64), lambda b, s: (0, 0)),
            pl.BlockSpec((1, 64), lambda b, s: (0, 0)),
            pl.BlockSpec((64, 64), lambda b, s: (0, 0)),
            pl.BlockSpec((1, 64), lambda b, s: (0, 0)),
            pl.BlockSpec((64, 128), lambda b, s: (0, 0)),
            pl.BlockSpec((1, 128), lambda b, s: (0, 0)),
        ],
        out_specs=pl.BlockSpec((1, SBLK, 128), lambda b, s: (b, s, 0)),
        out_shape=jax.ShapeDtypeStruct((B, S, 128), jnp.float32),
    )(G, c, w1, a1, w2, a2, w3, a3)


def kernel(xyz, points, W1, b1, g1, be1, W2, b2, g2, be2, W3, b3, g3, be3):
    nxyz, nxyzb, cc, xx, xyzb = _fps(xyz)

    xyz_t = jnp.transpose(xyz, (0, 2, 1))
    pts_t = jnp.transpose(points, (0, 2, 1))
    F = jnp.concatenate(
        [xyz_t, pts_t, jnp.zeros((B, N, DPAD - 67), jnp.float32)], axis=-1)

    G = _sc_group(xyzb.reshape(-1), xx.reshape(-1), nxyzb.reshape(-1),
                  cc.reshape(-1), F).reshape(B, S, NS, DPAD)

    w1 = jnp.pad(W1 * g1[:, None], ((0, 0), (0, DPAD - 67))).T
    a1 = (b1 * g1 + be1)[None, :]
    w2 = (W2 * g2[:, None]).T
    a2 = (b2 * g2 + be2)[None, :]
    w3 = (W3 * g3[:, None]).T
    a3 = (b3 * g3 + be3)[None, :]

    c = jnp.transpose(nxyz, (1, 2, 0))
    out = _mlp(G, c, w1, a1, w2, a2, w3, a3)

    new_xyz = jnp.transpose(nxyz, (1, 0, 2))
    new_points = jnp.transpose(out, (0, 2, 1))
    return new_xyz, new_points


# speedup vs baseline: 418.8156x; 1.0051x over previous
import functools

import jax
import jax.numpy as jnp
from jax import lax
from jax.experimental import pallas as pl
from jax.experimental.pallas import tpu as pltpu
from jax.experimental.pallas import tpu_sc as plsc

B = 8
N = 4096
S = 512
NS = 32
R2 = 0.2 ** 2
DPAD = 128
NTILES = 32
S_PER_TILE = S * B // NTILES
NCHUNK = N // 16


def _r(v):
    return v.astype(jnp.bfloat16).astype(jnp.float32)


def _fps_body(xyz_ref, fars_ref, xx_ref, xyzb_ref):
    x = xyz_ref[:, 0, :]
    y = xyz_ref[:, 1, :]
    z = xyz_ref[:, 2, :]
    xx_ref[...] = (x * x + y * y) + z * z
    xyzb_ref[0] = _r(x)
    xyzb_ref[1] = _r(y)
    xyzb_ref[2] = _r(z)
    iota = lax.broadcasted_iota(jnp.int32, (B, N), 1)
    iota_s = lax.broadcasted_iota(jnp.int32, (B, S), 1)

    def body(i, st):
        dist, far, fars = st
        farf = jnp.broadcast_to(far.astype(jnp.float32), (B, S))
        fars = jnp.where(iota_s == i, farf, fars)
        sel = iota == far
        cx = jnp.sum(jnp.where(sel, x, 0.0), axis=1, keepdims=True)
        cy = jnp.sum(jnp.where(sel, y, 0.0), axis=1, keepdims=True)
        cz = jnp.sum(jnp.where(sel, z, 0.0), axis=1, keepdims=True)
        dx, dy, dz = x - cx, y - cy, z - cz
        d = (dx * dx + dy * dy) + dz * dz
        dist = jnp.minimum(dist, d)
        mx = jnp.max(dist, axis=1, keepdims=True)
        far = jnp.min(jnp.where(dist == mx, iota, N), axis=1, keepdims=True)
        return dist, far, fars

    dist0 = jnp.full((B, N), 1e10, jnp.float32)
    far0 = jnp.zeros((B, 1), jnp.int32)
    f0 = x[:, 0:S] * 0.0
    _, _, fars = lax.fori_loop(0, S, body, (dist0, far0, f0))
    fars_ref[...] = fars.astype(jnp.int32)


def _fps(xyz):
    return pl.pallas_call(
        _fps_body,
        out_shape=(
            jax.ShapeDtypeStruct((B, S), jnp.int32),
            jax.ShapeDtypeStruct((B, N), jnp.float32),
            jax.ShapeDtypeStruct((3, B, N), jnp.float32),
        ),
    )(xyz)


GB = 4
NGB = S_PER_TILE // GB
UNROLL = 16


def _sc_group_body(xyzb_hbm, xyzf_hbm, xx_hbm, fars_hbm, f_hbm,
                   out_hbm, nxyz_hbm,
                   xv, yv, zv, xfv, yfv, zfv, xxv, farsv, nxv, nyv, nzv,
                   idxbuf, idxsel, gbuf, gsem0, gsem1, osem0, osem1):
    cid = lax.axis_index("c")
    sid = lax.axis_index("s")
    wid = sid * 2 + cid
    b = wid // (NTILES // B)
    q = wid % (NTILES // B)
    s0 = q * S_PER_TILE

    pltpu.sync_copy(xyzb_hbm.at[pl.ds((0 * B + b) * N, N)], xv)
    pltpu.sync_copy(xyzb_hbm.at[pl.ds((1 * B + b) * N, N)], yv)
    pltpu.sync_copy(xyzb_hbm.at[pl.ds((2 * B + b) * N, N)], zv)
    pltpu.sync_copy(xyzf_hbm.at[pl.ds((b * 3 + 0) * N, N)], xfv)
    pltpu.sync_copy(xyzf_hbm.at[pl.ds((b * 3 + 1) * N, N)], yfv)
    pltpu.sync_copy(xyzf_hbm.at[pl.ds((b * 3 + 2) * N, N)], zfv)
    pltpu.sync_copy(xx_hbm.at[pl.ds(b * N, N)], xxv)
    pltpu.sync_copy(fars_hbm.at[pl.ds(b * S + s0, S_PER_TILE)], farsv)

    lane = lax.iota(jnp.int32, 16)
    lane0 = lane == 0
    gsems = (gsem0, gsem1)
    osems = (osem0, osem1)

    def scan_batch(k, p):
        for j in range(GB):
            ci = k * GB + j
            ii = jnp.full((16,), ci, jnp.int32)
            fi = plsc.load_gather(farsv, [ii])
            cxs = plsc.load_gather(xv, [fi])
            cys = plsc.load_gather(yv, [fi])
            czs = plsc.load_gather(zv, [fi])
            cxf = plsc.load_gather(xfv, [fi])
            cyf = plsc.load_gather(yfv, [fi])
            czf = plsc.load_gather(zfv, [fi])
            ccs = (cxf * cxf + cyf * cyf) + czf * czf
            plsc.store_scatter(nxv, [ii], cxf, mask=lane0)
            plsc.store_scatter(nyv, [ii], cyf, mask=lane0)
            plsc.store_scatter(nzv, [ii], czf, mask=lane0)

            def cond(st):
                jj, cnt = st
                return (jj < NCHUNK // UNROLL) & (cnt < NS)

            def chunk(st):
                jj, cnt = st
                cntv = jnp.broadcast_to(cnt, (16,))
                masks = []
                for u in range(UNROLL):
                    off = (jj * UNROLL + u) * 16
                    xc = xv[pl.ds(off, 16)]
                    yc = yv[pl.ds(off, 16)]
                    zc = zv[pl.ds(off, 16)]
                    xxc = xxv[pl.ds(off, 16)]
                    dot = (cxs * xc + cys * yc) + czs * zc
                    d = (ccs + xxc) - 2.0 * dot
                    masks.append(d <= R2)
                offv = cntv
                for u in range(UNROLL):
                    m = masks[u]
                    off = (jj * UNROLL + u) * 16
                    pos = plsc.cumsum(m.astype(jnp.int32)) + offv
                    plsc.store_scatter(idxbuf, [pos - 1], lane + off, mask=m)
                    if u + 1 < UNROLL:
                        offv = offv + plsc.all_reduce_population_count(m)
                total = masks[0].astype(jnp.int32)
                for u in range(1, UNROLL):
                    total = total + masks[u].astype(jnp.int32)
                return jj + 1, cnt + jnp.sum(total)

            _, cnt = lax.while_loop(cond, chunk, (jnp.int32(0), jnp.int32(0)))

            first = plsc.load_gather(idxbuf, [jnp.zeros((16,), jnp.int32)])
            for h in range(NS // 16):
                v = idxbuf[pl.ds(h * 16, 16)]
                v = jnp.where(lane + h * 16 < cnt, v, first)
                idxsel[p, pl.ds(j * NS + h * 16, 16)] = v

    def gather_start(p):
        pltpu.async_copy(f_hbm.at[b].at[idxsel.at[p]], gbuf.at[p], gsems[p])

    def gather_wait(p):
        pltpu.make_async_copy(f_hbm.at[b].at[idxsel.at[p]], gbuf.at[p],
                              gsems[p]).wait()

    def out_dst(k):
        return out_hbm.at[b, pl.ds((s0 + k * GB) * NS, GB * NS)]

    def outcopy_start(k, p):
        pltpu.async_copy(gbuf.at[p], out_dst(k), osems[p])

    def outcopy_wait(k, p):
        pltpu.make_async_copy(gbuf.at[p], out_dst(k), osems[p]).wait()

    def pair(t, carry):
        k0 = 2 * t

        @pl.when(t >= 1)
        def _():
            outcopy_wait(k0, 0)
        scan_batch(k0, 0)

        @pl.when(t >= 1)
        def _():
            gather_wait(1)
            outcopy_start(k0 - 1, 1)
        gather_start(0)

        k1 = 2 * t + 1

        @pl.when(t >= 1)
        def _():
            outcopy_wait(k1, 1)
        scan_batch(k1, 1)

        gather_wait(0)
        outcopy_start(k0, 0)
        gather_start(1)
        return carry

    lax.fori_loop(0, NGB // 2, pair, jnp.int32(0))

    k_last = NGB - 1
    gather_wait(1)
    outcopy_start(k_last, 1)
    outcopy_wait(k_last - 1, 0)
    outcopy_wait(k_last, 1)

    pltpu.sync_copy(nxv, nxyz_hbm.at[pl.ds((0 * B + b) * S + s0, S_PER_TILE)])
    pltpu.sync_copy(nyv, nxyz_hbm.at[pl.ds((1 * B + b) * S + s0, S_PER_TILE)])
    pltpu.sync_copy(nzv, nxyz_hbm.at[pl.ds((2 * B + b) * S + s0, S_PER_TILE)])


@functools.partial(
    pl.kernel,
    out_type=(
        jax.ShapeDtypeStruct((B, S * NS, DPAD), jnp.float32),
        jax.ShapeDtypeStruct((3 * B * S,), jnp.float32),
    ),
    mesh=plsc.VectorSubcoreMesh(
        core_axis_name="c", subcore_axis_name="s", num_cores=2, num_subcores=16
    ),
    compiler_params=pltpu.CompilerParams(needs_layout_passes=False),
    scratch_types=[
        pltpu.VMEM((N,), jnp.float32),
        pltpu.VMEM((N,), jnp.float32),
        pltpu.VMEM((N,), jnp.float32),
        pltpu.VMEM((N,), jnp.float32),
        pltpu.VMEM((N,), jnp.float32),
        pltpu.VMEM((N,), jnp.float32),
        pltpu.VMEM((N,), jnp.float32),
        pltpu.VMEM((S_PER_TILE,), jnp.int32),
        pltpu.VMEM((S_PER_TILE,), jnp.float32),
        pltpu.VMEM((S_PER_TILE,), jnp.float32),
        pltpu.VMEM((S_PER_TILE,), jnp.float32),
        pltpu.VMEM((NS + 16 * UNROLL,), jnp.int32),
        pltpu.VMEM((2, GB * NS), jnp.int32),
        pltpu.VMEM((2, GB * NS, DPAD), jnp.float32),
        pltpu.SemaphoreType.DMA,
        pltpu.SemaphoreType.DMA,
        pltpu.SemaphoreType.DMA,
        pltpu.SemaphoreType.DMA,
    ],
)
def _sc_group(*refs):
    _sc_group_body(*refs)


SBLK = 128


def _mlp_body(g_ref, c_ref, w1_ref, a1_ref, w2_ref, a2_ref, w3_ref, a3_ref,
              out_ref):
    X = g_ref[0].reshape(SBLK * NS, DPAD)
    h = jnp.dot(X, w1_ref[...], preferred_element_type=jnp.float32)
    h = h + a1_ref[...]
    corr = jnp.dot(c_ref[0], w1_ref[0:3, :], preferred_element_type=jnp.float32)
    h = h.reshape(SBLK, NS, 64) - corr[:, None, :]
    h = jnp.maximum(h, 0.0).reshape(SBLK * NS, 64)
    h = jnp.dot(h, w2_ref[...], preferred_element_type=jnp.float32) + a2_ref[...]
    h = jnp.maximum(h, 0.0)
    h = jnp.dot(h, w3_ref[...], preferred_element_type=jnp.float32) + a3_ref[...]
    h = jnp.maximum(h, 0.0)
    out_ref[0] = jnp.max(h.reshape(SBLK, NS, 128), axis=1)


def _mlp(G, c, w1, a1, w2, a2, w3, a3):
    grid = (B, S // SBLK)
    return pl.pallas_call(
        _mlp_body,
        grid=grid,
        in_specs=[
            pl.BlockSpec((1, SBLK, NS, DPAD), lambda b, s: (b, s, 0, 0)),
            pl.BlockSpec((1, SBLK, 3), lambda b, s: (b, s, 0)),
            pl.BlockSpec((DPAD, 64), lambda b, s: (0, 0)),
            pl.BlockSpec((1, 64), lambda b, s: (0, 0)),
            pl.BlockSpec((64, 64), lambda b, s: (0, 0)),
            pl.BlockSpec((1, 64), lambda b, s: (0, 0)),
            pl.BlockSpec((64, 128), lambda b, s: (0, 0)),
            pl.BlockSpec((1, 128), lambda b, s: (0, 0)),
        ],
        out_specs=pl.BlockSpec((1, SBLK, 128), lambda b, s: (b, s, 0)),
        out_shape=jax.ShapeDtypeStruct((B, S, 128), jnp.float32),
    )(G, c, w1, a1, w2, a2, w3, a3)


def kernel(xyz, points, W1, b1, g1, be1, W2, b2, g2, be2, W3, b3, g3, be3):
    fars, xx, xyzb = _fps(xyz)

    xyz_t = jnp.transpose(xyz, (0, 2, 1))
    pts_t = jnp.transpose(points, (0, 2, 1))
    F = jnp.concatenate(
        [xyz_t, pts_t, jnp.zeros((B, N, DPAD - 67), jnp.float32)], axis=-1)

    G, nxyz_flat = _sc_group(xyzb.reshape(-1), xyz.reshape(-1),
                             xx.reshape(-1), fars.reshape(-1), F)
    G = G.reshape(B, S, NS, DPAD)
    nxyz = nxyz_flat.reshape(3, B, S)

    w1 = jnp.pad(W1 * g1[:, None], ((0, 0), (0, DPAD - 67))).T
    a1 = (b1 * g1 + be1)[None, :]
    w2 = (W2 * g2[:, None]).T
    a2 = (b2 * g2 + be2)[None, :]
    w3 = (W3 * g3[:, None]).T
    a3 = (b3 * g3 + be3)[None, :]

    c = jnp.transpose(nxyz, (1, 2, 0))
    out = _mlp(G, c, w1, a1, w2, a2, w3, a3)

    new_xyz = jnp.transpose(nxyz, (1, 0, 2))
    new_points = jnp.transpose(out, (0, 2, 1))
    return new_xyz, new_points
